# Initial kernel scaffold; baseline (speedup 1.0000x reference)
import functools

import jax
import jax.numpy as jnp
from jax import lax
from jax.experimental import pallas as pl
from jax.experimental.pallas import tpu as pltpu
from jax.experimental.pallas import tpu_sc as plsc

D = 128
SEQ = 200
TOK = 1024 * SEQ
NC, NS = 2, 16
NW = NC * NS
TPW = TOK // NW
C = 128
NCHUNK = TPW // C
LN_EPS = 1e-12
KVEC = D // 16


def _ln_body(ids, tts, word, pos, typ, gamma, beta, out,
             comb, typev, rows, idxv, ttv, gv, bv, sem):
    wid = lax.axis_index("s") * NC + lax.axis_index("c")
    base0 = wid * TPW

    pltpu.sync_copy(pos.at[pl.ds(0, SEQ)], comb.at[pl.ds(0, SEQ)])
    pltpu.sync_copy(pos.at[pl.ds(0, SEQ)], comb.at[pl.ds(SEQ, SEQ)])
    pltpu.sync_copy(typ, typev)
    pltpu.sync_copy(gamma, gv)
    pltpu.sync_copy(beta, bv)

    def add_type(l, carry):
        for k in range(KVEC):
            s = pl.ds(k * 16, 16)
            comb[l, s] = comb[l, s] + typev[0, s]
            comb[SEQ + l, s] = comb[SEQ + l, s] + typev[1, s]
        return carry

    lax.fori_loop(0, SEQ, add_type, 0)

    lanes = lax.iota(jnp.int32, 16)

    def chunk_body(c, carry):
        base = base0 + c * C
        pltpu.sync_copy(ids.at[pl.ds(base, C)], idxv)
        pltpu.sync_copy(tts.at[pl.ds(base, C)], ttv)
        pltpu.async_copy(word.at[idxv], rows, sem).wait()
        pos0 = lax.rem(base, SEQ)

        def tok_body(t, tc):
            tsplat = jnp.full((16,), t, jnp.int32)
            ttk = plsc.load_gather(ttv, [tsplat])
            p = lax.rem(pos0 + t, SEQ)
            arow = ttk * SEQ + p
            xs = []
            for k in range(KVEC):
                ad = plsc.load_gather(comb, [arow, lanes + (k * 16)])
                xs.append(rows[t, pl.ds(k * 16, 16)] + ad)
            s = xs[0]
            for k in range(1, KVEC):
                s = s + xs[k]
            sq = xs[0] * xs[0]
            for k in range(1, KVEC):
                sq = sq + xs[k] * xs[k]
            totv = jnp.full((16,), jnp.sum(s))
            tot2v = jnp.full((16,), jnp.sum(sq))
            meanv = totv * (1.0 / D)
            varv = tot2v * (1.0 / D) - meanv * meanv + LN_EPS
            yi = jnp.int32(0x5F3759DF) - lax.shift_right_arithmetic(
                plsc.bitcast(varv, jnp.int32), 1)
            y = plsc.bitcast(yi, jnp.float32)
            for _ in range(3):
                y = y * (1.5 - 0.5 * varv * y * y)
            for k in range(KVEC):
                s16 = pl.ds(k * 16, 16)
                rows[t, s16] = (xs[k] - meanv) * (gv[s16] * y) + bv[s16]
            return tc

        lax.fori_loop(0, C, tok_body, 0)
        pltpu.sync_copy(rows, out.at[pl.ds(base, C)])
        return carry

    lax.fori_loop(0, NCHUNK, chunk_body, 0)


_ln_kernel = functools.partial(
    pl.kernel,
    out_type=jax.ShapeDtypeStruct((TOK, D), jnp.float32),
    mesh=plsc.VectorSubcoreMesh(core_axis_name="c", subcore_axis_name="s"),
    scratch_types=[
        pltpu.VMEM((2 * SEQ, D), jnp.float32),
        pltpu.VMEM((2, D), jnp.float32),
        pltpu.VMEM((C, D), jnp.float32),
        pltpu.VMEM((C,), jnp.int32),
        pltpu.VMEM((C,), jnp.int32),
        pltpu.VMEM((D,), jnp.float32),
        pltpu.VMEM((D,), jnp.float32),
        pltpu.SemaphoreType.DMA,
    ],
)(_ln_body)


def kernel(input_ids, token_type_ids, word_emb, pos_emb, type_emb, gamma, beta):
    b, l = input_ids.shape
    ids = input_ids.reshape(-1).astype(jnp.int32)
    tts = token_type_ids.reshape(-1).astype(jnp.int32)
    out = _ln_kernel(ids, tts, word_emb, pos_emb, type_emb, gamma, beta)
    return out.reshape(b, l, D)


# speedup vs baseline: 1.8941x; 1.8941x over previous
import functools

import jax
import jax.numpy as jnp
from jax import lax
from jax.experimental import pallas as pl
from jax.experimental.pallas import tpu as pltpu
from jax.experimental.pallas import tpu_sc as plsc

D = 128
SEQ = 200
TOK = 1024 * SEQ
NC, NS = 2, 16
NW = NC * NS
TPW = TOK // NW
C = 128
NCHUNK = TPW // C
LN_EPS = 1e-12
KVEC = D // 16


def _ln_body(ids, tts, word, pos, typ, gamma, beta, out,
             comb, typev, rows, idxv, ttv, gv, bv, sem):
    wid = lax.axis_index("s") * NC + lax.axis_index("c")
    base0 = wid * TPW

    pltpu.sync_copy(pos.at[pl.ds(0, SEQ)], comb.at[pl.ds(0, SEQ)])
    pltpu.sync_copy(pos.at[pl.ds(0, SEQ)], comb.at[pl.ds(SEQ, SEQ)])
    pltpu.sync_copy(typ, typev)
    pltpu.sync_copy(gamma, gv)
    pltpu.sync_copy(beta, bv)

    def add_type(l, carry):
        for k in range(KVEC):
            s = pl.ds(k * 16, 16)
            comb[l, s] = comb[l, s] + typev[0, s]
            comb[SEQ + l, s] = comb[SEQ + l, s] + typev[1, s]
        return carry

    lax.fori_loop(0, SEQ, add_type, 0)

    lanes = lax.iota(jnp.int32, 16)

    def chunk_body(c, carry):
        base = base0 + c * C
        pltpu.sync_copy(ids.at[pl.ds(base, C)], idxv)
        pltpu.sync_copy(tts.at[pl.ds(base, C)], ttv)
        pltpu.async_copy(word.at[idxv], rows, sem).wait()
        pos0 = lax.rem(base, SEQ)

        def tok_body(t, tc):
            tsplat = jnp.full((16,), t, jnp.int32)
            ttk = plsc.load_gather(ttv, [tsplat])
            p = lax.rem(pos0 + t, SEQ)
            arow = ttk * SEQ + p
            xs = []
            for k in range(KVEC):
                ad = plsc.load_gather(comb, [arow, lanes + (k * 16)])
                xs.append(rows[t, pl.ds(k * 16, 16)] + ad)
            s = xs[0]
            for k in range(1, KVEC):
                s = s + xs[k]
            sq = xs[0] * xs[0]
            for k in range(1, KVEC):
                sq = sq + xs[k] * xs[k]
            totv = jnp.full((16,), jnp.sum(s))
            tot2v = jnp.full((16,), jnp.sum(sq))
            meanv = totv * (1.0 / D)
            varv = tot2v * (1.0 / D) - meanv * meanv + LN_EPS
            yi = jnp.int32(0x5F3759DF) - lax.shift_right_arithmetic(
                plsc.bitcast(varv, jnp.int32), 1)
            y = plsc.bitcast(yi, jnp.float32)
            for _ in range(3):
                y = y * (1.5 - 0.5 * varv * y * y)
            for k in range(KVEC):
                s16 = pl.ds(k * 16, 16)
                rows[t, s16] = (xs[k] - meanv) * (gv[s16] * y) + bv[s16]
            return tc

        lax.fori_loop(0, C, tok_body, 0)
        pltpu.sync_copy(rows, out.at[pl.ds(base, C)])
        return carry

    lax.fori_loop(0, NCHUNK, chunk_body, 0)


_ln_kernel = functools.partial(
    pl.kernel,
    out_type=jax.ShapeDtypeStruct((TOK, D), jnp.float32),
    mesh=plsc.VectorSubcoreMesh(core_axis_name="c", subcore_axis_name="s"),
    scratch_types=[
        pltpu.VMEM((2 * SEQ, D), jnp.float32),
        pltpu.VMEM((2, D), jnp.float32),
        pltpu.VMEM((C, D), jnp.float32),
        pltpu.VMEM((C,), jnp.int32),
        pltpu.VMEM((C,), jnp.int32),
        pltpu.VMEM((D,), jnp.float32),
        pltpu.VMEM((D,), jnp.float32),
        pltpu.SemaphoreType.DMA,
    ],
    compiler_params=pltpu.CompilerParams(needs_layout_passes=False),
)(_ln_body)


def kernel(input_ids, token_type_ids, word_emb, pos_emb, type_emb, gamma, beta):
    b, l = input_ids.shape
    ids = input_ids.reshape(-1).astype(jnp.int32)
    tts = token_type_ids.reshape(-1).astype(jnp.int32)
    out = _ln_kernel(ids, tts, word_emb, pos_emb, type_emb, gamma, beta)
    return out.reshape(b, l, D)


# speedup vs baseline: 4.0175x; 2.1210x over previous
import functools

import jax
import jax.numpy as jnp
from jax import lax
from jax.experimental import pallas as pl
from jax.experimental.pallas import tpu as pltpu
from jax.experimental.pallas import tpu_sc as plsc

D = 128
SEQ = 200
TOK = 1024 * SEQ
NC, NS = 2, 16
NW = NC * NS
TPW = TOK // NW
C = 128
NCHUNK = TPW // C
LN_EPS = 1e-12
KVEC = D // 16


def _ln_body(ids, tts, word, pos, typ, gamma, beta, out,
             comb, typev, rows, idxv, ttv, gv, bv, sem):
    wid = lax.axis_index("s") * NC + lax.axis_index("c")
    base0 = wid * TPW

    pltpu.sync_copy(pos.at[pl.ds(0, SEQ)], comb.at[pl.ds(0, SEQ)])
    pltpu.sync_copy(pos.at[pl.ds(0, SEQ)], comb.at[pl.ds(SEQ, SEQ)])
    pltpu.sync_copy(typ, typev)
    pltpu.sync_copy(gamma, gv)
    pltpu.sync_copy(beta, bv)

    def add_type(l, carry):
        for k in range(KVEC):
            s = pl.ds(k * 16, 16)
            comb[l, s] = comb[l, s] + typev[0, s]
            comb[SEQ + l, s] = comb[SEQ + l, s] + typev[1, s]
        return carry

    lax.fori_loop(0, SEQ, add_type, 0)

    lanes = lax.iota(jnp.int32, 16)

    def chunk_body(c, carry):
        base = base0 + c * C
        pltpu.sync_copy(ids.at[pl.ds(base, C)], idxv)
        pltpu.sync_copy(tts.at[pl.ds(base, C)], ttv)
        pltpu.async_copy(word.at[idxv], rows, sem).wait()
        pos0 = lax.rem(base, SEQ)

        @plsc.parallel_loop(0, C, 1, unroll=4)
        def tok_body(t):
            tsplat = jnp.full((16,), t, jnp.int32)
            ttk = plsc.load_gather(ttv, [tsplat])
            p = lax.rem(pos0 + t, SEQ)
            arow = ttk * SEQ + p
            xs = []
            for k in range(KVEC):
                ad = plsc.load_gather(comb, [arow, lanes + (k * 16)])
                xs.append(rows[t, pl.ds(k * 16, 16)] + ad)
            s = xs[0]
            for k in range(1, KVEC):
                s = s + xs[k]
            sq = xs[0] * xs[0]
            for k in range(1, KVEC):
                sq = sq + xs[k] * xs[k]
            totv = jnp.full((16,), jnp.sum(s))
            tot2v = jnp.full((16,), jnp.sum(sq))
            meanv = totv * (1.0 / D)
            varv = tot2v * (1.0 / D) - meanv * meanv + LN_EPS
            yi = jnp.int32(0x5F3759DF) - lax.shift_right_arithmetic(
                plsc.bitcast(varv, jnp.int32), 1)
            y = plsc.bitcast(yi, jnp.float32)
            for _ in range(3):
                y = y * (1.5 - 0.5 * varv * y * y)
            for k in range(KVEC):
                s16 = pl.ds(k * 16, 16)
                rows[t, s16] = (xs[k] - meanv) * (gv[s16] * y) + bv[s16]

        pltpu.sync_copy(rows, out.at[pl.ds(base, C)])
        return carry

    lax.fori_loop(0, NCHUNK, chunk_body, 0)


_ln_kernel = functools.partial(
    pl.kernel,
    out_type=jax.ShapeDtypeStruct((TOK, D), jnp.float32),
    mesh=plsc.VectorSubcoreMesh(core_axis_name="c", subcore_axis_name="s"),
    scratch_types=[
        pltpu.VMEM((2 * SEQ, D), jnp.float32),
        pltpu.VMEM((2, D), jnp.float32),
        pltpu.VMEM((C, D), jnp.float32),
        pltpu.VMEM((C,), jnp.int32),
        pltpu.VMEM((C,), jnp.int32),
        pltpu.VMEM((D,), jnp.float32),
        pltpu.VMEM((D,), jnp.float32),
        pltpu.SemaphoreType.DMA,
    ],
    compiler_params=pltpu.CompilerParams(needs_layout_passes=False),
)(_ln_body)


def kernel(input_ids, token_type_ids, word_emb, pos_emb, type_emb, gamma, beta):
    b, l = input_ids.shape
    ids = input_ids.reshape(-1).astype(jnp.int32)
    tts = token_type_ids.reshape(-1).astype(jnp.int32)
    out = _ln_kernel(ids, tts, word_emb, pos_emb, type_emb, gamma, beta)
    return out.reshape(b, l, D)
